# baseline (speedup 1.0000x reference)
import functools

import jax
import jax.numpy as jnp
from jax import lax
from jax.experimental import pallas as pl
from jax.experimental.pallas import tpu as pltpu
from jax.experimental.pallas import tpu_sc as plsc

_N = 50000
_E = 800000
_G = 128

_NSUB = 16
_ZROWS = 3128
_ACC_ROWS = _ZROWS * _NSUB
_NPAD = _ACC_ROWS
_NP = _NPAD // 4
_BP = 3128
_GRID = _NP // _BP
_EW = 384
_CH = 6
_EROWS = 2112
_EPAD = _EROWS * _EW


def _f32(x):
    return x.astype(jnp.float32)


def _blockdiag4(w):
    return jnp.kron(jnp.eye(4, dtype=jnp.float32), w)


def _embed_body(xp_ref, wa_ref, wb_ref, b_ref, out_ref):
    i16 = lax.broadcasted_iota(jnp.int32, (_BP, 16), 1)
    i8 = lax.broadcasted_iota(jnp.int32, (_BP, 8), 1)
    ohs = jnp.concatenate(
        [_f32(xp_ref[:, 2 * j:2 * j + 1] == i16) for j in range(4)], axis=1)
    ohc = jnp.concatenate(
        [_f32(xp_ref[:, 2 * j + 1:2 * j + 2] == i8) for j in range(4)], axis=1)
    z = (jnp.dot(ohs, wa_ref[...], preferred_element_type=jnp.float32)
         + jnp.dot(ohc, wb_ref[...], preferred_element_type=jnp.float32)
         + b_ref[...])
    out_ref[...] = jnp.maximum(z, 0.0)


def _embed(xp, shape_emb, color_emb, lin_W, lin_b):
    wa = _blockdiag4(shape_emb @ lin_W[:, 0:8].T)
    wb = _blockdiag4(color_emb @ lin_W[:, 8:16].T)
    bp = jnp.tile(lin_b, 4).reshape(1, 128)
    return pl.pallas_call(
        _embed_body,
        grid=(_GRID,),
        in_specs=[
            pl.BlockSpec((_BP, 8), lambda i: (i, 0)),
            pl.BlockSpec((64, 128), lambda i: (0, 0)),
            pl.BlockSpec((32, 128), lambda i: (0, 0)),
            pl.BlockSpec((1, 128), lambda i: (0, 0)),
        ],
        out_specs=pl.BlockSpec((_BP, 128), lambda i: (i, 0)),
        out_shape=jax.ShapeDtypeStruct((_NP, 128), jnp.float32),
    )(xp, wa, wb, bp)


def _mix1_body(aa_ref, ab_ref, h_ref, wl_ref, wr_ref, bl_ref, br_ref,
               outl_ref, outr_ref):
    cat = jnp.concatenate([aa_ref[...] + ab_ref[...], h_ref[...]], axis=1)
    zl = (jnp.dot(cat, wl_ref[...], preferred_element_type=jnp.float32)
          + bl_ref[...])
    zr = (jnp.dot(cat, wr_ref[...], preferred_element_type=jnp.float32)
          + br_ref[...])
    outl_ref[...] = jnp.maximum(zl, 0.0)
    outr_ref[...] = jnp.maximum(zr, 0.0)


def _mix1(aggA, aggB, h1, Wrel, brel, Wroot):
    wrel_t = Wrel.T
    wroot_t = Wroot.T
    wl = jnp.concatenate([_blockdiag4(wrel_t[:, 0:32]),
                          _blockdiag4(wroot_t[:, 0:32])], axis=0)
    wr = jnp.concatenate([_blockdiag4(wrel_t[:, 32:64]),
                          _blockdiag4(wroot_t[:, 32:64])], axis=0)
    bl = jnp.tile(brel[0:32], 4).reshape(1, 128)
    br = jnp.tile(brel[32:64], 4).reshape(1, 128)
    return pl.pallas_call(
        _mix1_body,
        grid=(_GRID,),
        in_specs=[
            pl.BlockSpec((_BP, 128), lambda i: (i, 0)),
            pl.BlockSpec((_BP, 128), lambda i: (i, 0)),
            pl.BlockSpec((_BP, 128), lambda i: (i, 0)),
            pl.BlockSpec((256, 128), lambda i: (0, 0)),
            pl.BlockSpec((256, 128), lambda i: (0, 0)),
            pl.BlockSpec((1, 128), lambda i: (0, 0)),
            pl.BlockSpec((1, 128), lambda i: (0, 0)),
        ],
        out_specs=[
            pl.BlockSpec((_BP, 128), lambda i: (i, 0)),
            pl.BlockSpec((_BP, 128), lambda i: (i, 0)),
        ],
        out_shape=[
            jax.ShapeDtypeStruct((_NP, 128), jnp.float32),
            jax.ShapeDtypeStruct((_NP, 128), jnp.float32),
        ],
    )(aggA, aggB, h1, wl, wr, bl, br)


def _pool_body(al_ref, ar_ref, hl_ref, hr_ref, w2_ref, b2_ref, bat_ref,
               wc_ref, bc_ref, sums_ref, cnt_ref, out_ref):
    @pl.when(pl.program_id(0) == 0)
    def _():
        sums_ref[...] = jnp.zeros_like(sums_ref)
        cnt_ref[...] = jnp.zeros_like(cnt_ref)

    ig = lax.broadcasted_iota(jnp.int32, (_BP, _G), 1)
    dnt = (((0,), (0,)), ((), ()))
    ones = jnp.ones((_BP, 8), jnp.float32)
    for j in range(4):
        cat = jnp.concatenate(
            [al_ref[:, 32 * j:32 * j + 32], ar_ref[:, 32 * j:32 * j + 32],
             hl_ref[:, 32 * j:32 * j + 32], hr_ref[:, 32 * j:32 * j + 32]],
            axis=1)
        z = (jnp.dot(cat, w2_ref[...], preferred_element_type=jnp.float32)
             + b2_ref[...])
        h3 = jnp.maximum(z, 0.0)
        oh = _f32(bat_ref[:, j:j + 1] == ig)
        sums_ref[...] += lax.dot_general(oh, h3, dnt,
                                         preferred_element_type=jnp.float32)
        cnt_ref[...] += lax.dot_general(oh, ones, dnt,
                                        preferred_element_type=jnp.float32)

    @pl.when(pl.program_id(0) == _GRID - 1)
    def _():
        denom = jnp.maximum(cnt_ref[:, 0:1], 1.0)
        pooled = sums_ref[...] / denom
        out_ref[...] = (jnp.dot(pooled, wc_ref[...],
                                preferred_element_type=jnp.float32)
                        + bc_ref[...])


def _pool_cls(agg_l, agg_r, h_l, h_r, Wrel, brel, Wroot, batp, cls_W, cls_b):
    nc = cls_W.shape[0]
    w2 = jnp.concatenate([Wrel.T, Wroot.T], axis=0)
    b2 = brel.reshape(1, 64)
    wc = cls_W.T
    bc = cls_b.reshape(1, nc)
    outs = pl.pallas_call(
        _pool_body,
        grid=(_GRID,),
        in_specs=[
            pl.BlockSpec((_BP, 128), lambda i: (i, 0)),
            pl.BlockSpec((_BP, 128), lambda i: (i, 0)),
            pl.BlockSpec((_BP, 128), lambda i: (i, 0)),
            pl.BlockSpec((_BP, 128), lambda i: (i, 0)),
            pl.BlockSpec((128, 64), lambda i: (0, 0)),
            pl.BlockSpec((1, 64), lambda i: (0, 0)),
            pl.BlockSpec((_BP, 4), lambda i: (i, 0)),
            pl.BlockSpec((64, nc), lambda i: (0, 0)),
            pl.BlockSpec((1, nc), lambda i: (0, 0)),
        ],
        out_specs=[
            pl.BlockSpec((_G, 64), lambda i: (0, 0)),
            pl.BlockSpec((_G, 8), lambda i: (0, 0)),
            pl.BlockSpec((_G, nc), lambda i: (0, 0)),
        ],
        out_shape=[
            jax.ShapeDtypeStruct((_G, 64), jnp.float32),
            jax.ShapeDtypeStruct((_G, 8), jnp.float32),
            jax.ShapeDtypeStruct((_G, nc), jnp.float32),
        ],
    )(agg_l, agg_r, h_l, h_r, w2, b2, batp, wc, bc)
    return outs[2]


@functools.lru_cache(maxsize=None)
def _make_edge_agg(core_rows, sub_rows):
    nch = sub_rows // _CH
    mesh = plsc.VectorSubcoreMesh(core_axis_name="c", subcore_axis_name="s")

    @functools.partial(
        pl.kernel,
        out_type=[jax.ShapeDtypeStruct((_NPAD, 32), jnp.float32),
                  jax.ShapeDtypeStruct((_NPAD, 32), jnp.float32)],
        mesh=mesh,
        compiler_params=pltpu.CompilerParams(use_tc_tiling_on_sc=False),
        scratch_types=[
            pltpu.VMEM((_CH, _EW), jnp.int32),
            pltpu.VMEM((_CH, _EW), jnp.int32),
            pltpu.VMEM((_EW, 32), jnp.float32),
            pltpu.VMEM((_EW, 32), jnp.float32),
            pltpu.SemaphoreType.DMA,
            pltpu.SemaphoreType.DMA,
            pltpu.VMEM_SHARED((_ACC_ROWS, 32), jnp.float32),
        ],
    )
    def agg(ha, hb, src_hbm, dst_hbm, z_hbm, outa, outb,
            idx_s, idx_d, rows0, rows1, sem0, sem1, acc):
        c = lax.axis_index("c")
        s = lax.axis_index("s")
        pltpu.sync_copy(z_hbm, acc.at[pl.ds(s * _ZROWS, _ZROWS)])
        plsc.subcore_barrier()

        base = c * core_rows + s * sub_rows

        def walk(h_ref):
            def outer(t, carry):
                r0 = base + t * _CH
                pltpu.sync_copy(src_hbm.at[pl.ds(r0, _CH)], idx_s)
                pltpu.sync_copy(dst_hbm.at[pl.ds(r0, _CH)], idx_d)
                pltpu.async_copy(h_ref.at[idx_s.at[0]], rows0, sem0)

                def pair(p, carry2):
                    k = 2 * p
                    pltpu.async_copy(h_ref.at[idx_s.at[k + 1]], rows1, sem1)
                    pltpu.make_async_copy(h_ref.at[idx_s.at[k]],
                                          rows0, sem0).wait()
                    pltpu.sync_copy(rows0, acc.at[idx_d.at[k]], add=True)

                    @pl.when(k + 2 < _CH)
                    def _():
                        pltpu.async_copy(h_ref.at[idx_s.at[k + 2]],
                                         rows0, sem0)

                    pltpu.make_async_copy(h_ref.at[idx_s.at[k + 1]],
                                          rows1, sem1).wait()
                    pltpu.sync_copy(rows1, acc.at[idx_d.at[k + 1]], add=True)
                    return carry2

                return lax.fori_loop(0, _CH // 2, pair, carry)

            lax.fori_loop(0, nch, outer, 0)

        @pl.when(c == 0)
        def _():
            walk(ha)

        @pl.when(c == 1)
        def _():
            walk(hb)

        plsc.subcore_barrier()

        for cc, oref in ((0, outa), (1, outb)):
            @pl.when(c == cc)
            def _(oref=oref):
                pltpu.sync_copy(acc.at[pl.ds(s * _ZROWS, _ZROWS)],
                                oref.at[pl.ds(s * _ZROWS, _ZROWS)])

    return agg


def kernel(x, edge_index, batch, shape_emb, color_emb, lin_W, lin_b,
           conv1_Wrel, conv1_brel, conv1_Wroot,
           conv2_Wrel, conv2_brel, conv2_Wroot,
           cls_W, cls_b):
    src = edge_index[0]
    dst = edge_index[1]
    pad = _EPAD - _E
    src_p = jnp.concatenate([src, jnp.zeros((pad,), src.dtype)]
                            ).reshape(_EROWS, _EW).astype(jnp.int32)
    dst_p = jnp.concatenate([dst, jnp.full((pad,), _N, dst.dtype)]
                            ).reshape(_EROWS, _EW).astype(jnp.int32)
    zeros = jnp.zeros((_ZROWS, 32), jnp.float32)
    npad = _NPAD - _N
    xp = jnp.concatenate(
        [x.astype(jnp.int32), jnp.full((npad, 2), -1, jnp.int32)]
    ).reshape(_NP, 8)
    batp = jnp.concatenate(
        [batch.astype(jnp.int32), jnp.full((npad,), -1, jnp.int32)]
    ).reshape(_NP, 4)

    h1 = _embed(xp, shape_emb, color_emb, lin_W, lin_b)

    h1v = h1.reshape(_NPAD, 32)
    aggA, aggB = _make_edge_agg(_EROWS // 2, _EROWS // (2 * _NSUB))(
        h1v, h1v, src_p, dst_p, zeros)
    h2_l, h2_r = _mix1(aggA.reshape(_NP, 128), aggB.reshape(_NP, 128), h1,
                       conv1_Wrel, conv1_brel, conv1_Wroot)

    agg_l, agg_r = _make_edge_agg(0, _EROWS // _NSUB)(
        h2_l.reshape(_NPAD, 32), h2_r.reshape(_NPAD, 32), src_p, dst_p, zeros)

    return _pool_cls(agg_l.reshape(_NP, 128), agg_r.reshape(_NP, 128),
                     h2_l, h2_r,
                     conv2_Wrel, conv2_brel, conv2_Wroot, batp, cls_W, cls_b)


# speedup vs baseline: 1.3617x; 1.3617x over previous
import functools

import jax
import jax.numpy as jnp
from jax import lax
from jax.experimental import pallas as pl
from jax.experimental.pallas import tpu as pltpu
from jax.experimental.pallas import tpu_sc as plsc

_N = 50000
_E = 800000
_G = 128

_NSUB = 16
_ZROWS = 3128
_ACC_ROWS = _ZROWS * _NSUB
_NPAD = _ACC_ROWS
_NP = _NPAD // 4
_BP = 3128
_GRID = _NP // _BP
_EW = 256
_CH = 14
_EROWS = 3136
_EPAD = _EROWS * _EW


def _f32(x):
    return x.astype(jnp.float32)


def _blockdiag4(w):
    return jnp.kron(jnp.eye(4, dtype=jnp.float32), w)


def _embed_body(xp_ref, wa_ref, wb_ref, b_ref, out_ref):
    i16 = lax.broadcasted_iota(jnp.int32, (_BP, 16), 1)
    i8 = lax.broadcasted_iota(jnp.int32, (_BP, 8), 1)
    ohs = jnp.concatenate(
        [_f32(xp_ref[:, 2 * j:2 * j + 1] == i16) for j in range(4)], axis=1)
    ohc = jnp.concatenate(
        [_f32(xp_ref[:, 2 * j + 1:2 * j + 2] == i8) for j in range(4)], axis=1)
    z = (jnp.dot(ohs, wa_ref[...], preferred_element_type=jnp.float32)
         + jnp.dot(ohc, wb_ref[...], preferred_element_type=jnp.float32)
         + b_ref[...])
    out_ref[...] = jnp.maximum(z, 0.0)


def _embed(xp, shape_emb, color_emb, lin_W, lin_b):
    wa = _blockdiag4(shape_emb @ lin_W[:, 0:8].T)
    wb = _blockdiag4(color_emb @ lin_W[:, 8:16].T)
    bp = jnp.tile(lin_b, 4).reshape(1, 128)
    return pl.pallas_call(
        _embed_body,
        grid=(_GRID,),
        in_specs=[
            pl.BlockSpec((_BP, 8), lambda i: (i, 0)),
            pl.BlockSpec((64, 128), lambda i: (0, 0)),
            pl.BlockSpec((32, 128), lambda i: (0, 0)),
            pl.BlockSpec((1, 128), lambda i: (0, 0)),
        ],
        out_specs=pl.BlockSpec((_BP, 128), lambda i: (i, 0)),
        out_shape=jax.ShapeDtypeStruct((_NP, 128), jnp.float32),
    )(xp, wa, wb, bp)


def _mix1_body(aa_ref, ab_ref, h_ref, wl_ref, wr_ref, bl_ref, br_ref,
               outl_ref, outr_ref):
    cat = jnp.concatenate([aa_ref[...] + ab_ref[...], h_ref[...]], axis=1)
    zl = (jnp.dot(cat, wl_ref[...], preferred_element_type=jnp.float32)
          + bl_ref[...])
    zr = (jnp.dot(cat, wr_ref[...], preferred_element_type=jnp.float32)
          + br_ref[...])
    outl_ref[...] = jnp.maximum(zl, 0.0)
    outr_ref[...] = jnp.maximum(zr, 0.0)


def _mix1(aggA, aggB, h1, Wrel, brel, Wroot):
    wrel_t = Wrel.T
    wroot_t = Wroot.T
    wl = jnp.concatenate([_blockdiag4(wrel_t[:, 0:32]),
                          _blockdiag4(wroot_t[:, 0:32])], axis=0)
    wr = jnp.concatenate([_blockdiag4(wrel_t[:, 32:64]),
                          _blockdiag4(wroot_t[:, 32:64])], axis=0)
    bl = jnp.tile(brel[0:32], 4).reshape(1, 128)
    br = jnp.tile(brel[32:64], 4).reshape(1, 128)
    return pl.pallas_call(
        _mix1_body,
        grid=(_GRID,),
        in_specs=[
            pl.BlockSpec((_BP, 128), lambda i: (i, 0)),
            pl.BlockSpec((_BP, 128), lambda i: (i, 0)),
            pl.BlockSpec((_BP, 128), lambda i: (i, 0)),
            pl.BlockSpec((256, 128), lambda i: (0, 0)),
            pl.BlockSpec((256, 128), lambda i: (0, 0)),
            pl.BlockSpec((1, 128), lambda i: (0, 0)),
            pl.BlockSpec((1, 128), lambda i: (0, 0)),
        ],
        out_specs=[
            pl.BlockSpec((_BP, 128), lambda i: (i, 0)),
            pl.BlockSpec((_BP, 128), lambda i: (i, 0)),
        ],
        out_shape=[
            jax.ShapeDtypeStruct((_NP, 128), jnp.float32),
            jax.ShapeDtypeStruct((_NP, 128), jnp.float32),
        ],
    )(aggA, aggB, h1, wl, wr, bl, br)


def _pool_body(al_ref, ar_ref, hl_ref, hr_ref, w2_ref, b2_ref, bat_ref,
               wc_ref, bc_ref, sums_ref, cnt_ref, out_ref):
    @pl.when(pl.program_id(0) == 0)
    def _():
        sums_ref[...] = jnp.zeros_like(sums_ref)
        cnt_ref[...] = jnp.zeros_like(cnt_ref)

    ig = lax.broadcasted_iota(jnp.int32, (_BP, _G), 1)
    dnt = (((0,), (0,)), ((), ()))
    ones = jnp.ones((_BP, 8), jnp.float32)
    for j in range(4):
        cat = jnp.concatenate(
            [al_ref[:, 32 * j:32 * j + 32], ar_ref[:, 32 * j:32 * j + 32],
             hl_ref[:, 32 * j:32 * j + 32], hr_ref[:, 32 * j:32 * j + 32]],
            axis=1)
        z = (jnp.dot(cat, w2_ref[...], preferred_element_type=jnp.float32)
             + b2_ref[...])
        h3 = jnp.maximum(z, 0.0)
        oh = _f32(bat_ref[:, j:j + 1] == ig)
        sums_ref[...] += lax.dot_general(oh, h3, dnt,
                                         preferred_element_type=jnp.float32)
        cnt_ref[...] += lax.dot_general(oh, ones, dnt,
                                        preferred_element_type=jnp.float32)

    @pl.when(pl.program_id(0) == _GRID - 1)
    def _():
        denom = jnp.maximum(cnt_ref[:, 0:1], 1.0)
        pooled = sums_ref[...] / denom
        out_ref[...] = (jnp.dot(pooled, wc_ref[...],
                                preferred_element_type=jnp.float32)
                        + bc_ref[...])


def _pool_cls(agg_l, agg_r, h_l, h_r, Wrel, brel, Wroot, batp, cls_W, cls_b):
    nc = cls_W.shape[0]
    w2 = jnp.concatenate([Wrel.T, Wroot.T], axis=0)
    b2 = brel.reshape(1, 64)
    wc = cls_W.T
    bc = cls_b.reshape(1, nc)
    outs = pl.pallas_call(
        _pool_body,
        grid=(_GRID,),
        in_specs=[
            pl.BlockSpec((_BP, 128), lambda i: (i, 0)),
            pl.BlockSpec((_BP, 128), lambda i: (i, 0)),
            pl.BlockSpec((_BP, 128), lambda i: (i, 0)),
            pl.BlockSpec((_BP, 128), lambda i: (i, 0)),
            pl.BlockSpec((128, 64), lambda i: (0, 0)),
            pl.BlockSpec((1, 64), lambda i: (0, 0)),
            pl.BlockSpec((_BP, 4), lambda i: (i, 0)),
            pl.BlockSpec((64, nc), lambda i: (0, 0)),
            pl.BlockSpec((1, nc), lambda i: (0, 0)),
        ],
        out_specs=[
            pl.BlockSpec((_G, 64), lambda i: (0, 0)),
            pl.BlockSpec((_G, 8), lambda i: (0, 0)),
            pl.BlockSpec((_G, nc), lambda i: (0, 0)),
        ],
        out_shape=[
            jax.ShapeDtypeStruct((_G, 64), jnp.float32),
            jax.ShapeDtypeStruct((_G, 8), jnp.float32),
            jax.ShapeDtypeStruct((_G, nc), jnp.float32),
        ],
    )(agg_l, agg_r, h_l, h_r, w2, b2, batp, wc, bc)
    return outs[2]


@functools.lru_cache(maxsize=None)
def _make_edge_agg(core_rows, sub_rows):
    nch = sub_rows // _CH
    mesh = plsc.VectorSubcoreMesh(core_axis_name="c", subcore_axis_name="s")

    @functools.partial(
        pl.kernel,
        out_type=[jax.ShapeDtypeStruct((_NPAD, 32), jnp.float32),
                  jax.ShapeDtypeStruct((_NPAD, 32), jnp.float32)],
        mesh=mesh,
        compiler_params=pltpu.CompilerParams(use_tc_tiling_on_sc=False),
        scratch_types=[
            pltpu.VMEM((_CH, _EW), jnp.int32),
            pltpu.VMEM((_CH, _EW), jnp.int32),
            pltpu.VMEM((_EW, 32), jnp.float32),
            pltpu.VMEM((_EW, 32), jnp.float32),
            pltpu.SemaphoreType.DMA,
            pltpu.SemaphoreType.DMA,
            pltpu.VMEM_SHARED((_ACC_ROWS, 32), jnp.float32),
        ],
    )
    def agg(ha, hb, src_hbm, dst_hbm, z_hbm, outa, outb,
            idx_s, idx_d, rows0, rows1, sem0, sem1, acc):
        c = lax.axis_index("c")
        s = lax.axis_index("s")
        pltpu.sync_copy(z_hbm, acc.at[pl.ds(s * _ZROWS, _ZROWS)])
        plsc.subcore_barrier()

        base = c * core_rows + s * sub_rows

        def walk(h_ref):
            def outer(t, carry):
                r0 = base + t * _CH
                pltpu.sync_copy(src_hbm.at[pl.ds(r0, _CH)], idx_s)
                pltpu.sync_copy(dst_hbm.at[pl.ds(r0, _CH)], idx_d)
                pltpu.async_copy(h_ref.at[idx_s.at[0]], rows0, sem0)

                def pair(p, carry2):
                    k = 2 * p
                    pltpu.async_copy(h_ref.at[idx_s.at[k + 1]], rows1, sem1)
                    pltpu.make_async_copy(h_ref.at[idx_s.at[k]],
                                          rows0, sem0).wait()
                    pltpu.sync_copy(rows0, acc.at[idx_d.at[k]], add=True)

                    @pl.when(k + 2 < _CH)
                    def _():
                        pltpu.async_copy(h_ref.at[idx_s.at[k + 2]],
                                         rows0, sem0)

                    pltpu.make_async_copy(h_ref.at[idx_s.at[k + 1]],
                                          rows1, sem1).wait()
                    pltpu.sync_copy(rows1, acc.at[idx_d.at[k + 1]], add=True)
                    return carry2

                return lax.fori_loop(0, _CH // 2, pair, carry)

            lax.fori_loop(0, nch, outer, 0)

        @pl.when(c == 0)
        def _():
            walk(ha)

        @pl.when(c == 1)
        def _():
            walk(hb)

        plsc.subcore_barrier()

        for cc, oref in ((0, outa), (1, outb)):
            @pl.when(c == cc)
            def _(oref=oref):
                pltpu.sync_copy(acc.at[pl.ds(s * _ZROWS, _ZROWS)],
                                oref.at[pl.ds(s * _ZROWS, _ZROWS)])

    return agg


def kernel(x, edge_index, batch, shape_emb, color_emb, lin_W, lin_b,
           conv1_Wrel, conv1_brel, conv1_Wroot,
           conv2_Wrel, conv2_brel, conv2_Wroot,
           cls_W, cls_b):
    src = edge_index[0]
    dst = edge_index[1]
    pad = _EPAD - _E
    src_p = jnp.concatenate([src, jnp.zeros((pad,), src.dtype)]
                            ).reshape(_EROWS, _EW).astype(jnp.int32)
    dst_p = jnp.concatenate([dst, jnp.full((pad,), _N, dst.dtype)]
                            ).reshape(_EROWS, _EW).astype(jnp.int32)
    zeros = jnp.zeros((_ZROWS, 32), jnp.float32)
    npad = _NPAD - _N
    xp = jnp.concatenate(
        [x.astype(jnp.int32), jnp.full((npad, 2), -1, jnp.int32)]
    ).reshape(_NP, 8)
    batp = jnp.concatenate(
        [batch.astype(jnp.int32), jnp.full((npad,), -1, jnp.int32)]
    ).reshape(_NP, 4)

    h1 = _embed(xp, shape_emb, color_emb, lin_W, lin_b)

    h1v = h1.reshape(_NPAD, 32)
    aggA, aggB = _make_edge_agg(_EROWS // 2, _EROWS // (2 * _NSUB))(
        h1v, h1v, src_p, dst_p, zeros)
    h2_l, h2_r = _mix1(aggA.reshape(_NP, 128), aggB.reshape(_NP, 128), h1,
                       conv1_Wrel, conv1_brel, conv1_Wroot)

    agg_l, agg_r = _make_edge_agg(0, _EROWS // _NSUB)(
        h2_l.reshape(_NPAD, 32), h2_r.reshape(_NPAD, 32), src_p, dst_p, zeros)

    return _pool_cls(agg_l.reshape(_NP, 128), agg_r.reshape(_NP, 128),
                     h2_l, h2_r,
                     conv2_Wrel, conv2_brel, conv2_Wroot, batp, cls_W, cls_b)
